# baseline (speedup 1.0000x reference)
import functools

import jax
import jax.numpy as jnp
from jax.experimental import pallas as pl
from jax.experimental.pallas import tpu as pltpu

_B = 1024
_D = 64
_L = 50
_H = 4
_HD = 16
_M = 64
_F = _H * _M
_TOPK = 5
_TILE = 2048
_NEG = -1e30


def _normalize_rows(x):
    sq = x * x
    v = sq
    w = _D
    while w > 1:
        w //= 2
        v = v[:, :w] + v[:, w : 2 * w]
    n = jnp.sqrt(v)
    return x / jnp.maximum(n, 1e-4)


def _normalize_cols(x):
    sq = x * x
    v = sq
    w = _D
    while w > 1:
        w //= 2
        v = v[:w, :] + v[w : 2 * w, :]
    n = jnp.sqrt(v)
    return x / jnp.maximum(n, 1e-4)


def _performer(x, proj):
    outs = []
    for h in range(_H):
        xh = x[:, h * _HD : (h + 1) * _HD].astype(jnp.bfloat16)
        ph = proj[h].astype(jnp.bfloat16)
        outs.append(jnp.dot(xh, ph, preferred_element_type=jnp.float32))
    return jax.nn.softplus(jnp.concatenate(outs, axis=1))


def _top5(s, idx_off):
    cols = jax.lax.broadcasted_iota(jnp.int32, s.shape, 1)
    vals, idxs = [], []
    for _ in range(_TOPK):
        m = jnp.max(s, axis=1, keepdims=True)
        am = jnp.argmax(s, axis=1).astype(jnp.int32).reshape(-1, 1)
        vals.append(m)
        idxs.append(am + idx_off)
        s = jnp.where(cols == am, _NEG, s)
    return vals, idxs


def _fused_kernel(sums_ref, mask_ref, k_ref, proj_ref, projt_ref,
                  vals_ref, idxs_ref,
                  phiq_ref, mv_ref, mi_ref, *, num_tiles, num_labels):
    pid = pl.program_id(0)

    @pl.when(pid == 0)
    def _init():
        denom = jnp.maximum(
            jnp.sum(mask_ref[...], axis=1, keepdims=True), 1.0)
        q = _normalize_rows(sums_ref[...] / denom)
        phiq_ref[...] = _performer(q, proj_ref[...]).astype(jnp.bfloat16)
        mv_ref[...] = jnp.full(mv_ref.shape, _NEG, jnp.float32)
        mi_ref[...] = jnp.zeros(mi_ref.shape, jnp.int32)

    wn = _normalize_cols(k_ref[...])
    heads = []
    for h in range(_H):
        wh = wn[h * _HD : (h + 1) * _HD, :].astype(jnp.bfloat16)
        pt = projt_ref[h].astype(jnp.bfloat16)
        heads.append(jnp.dot(pt, wh, preferred_element_type=jnp.float32))
    phi_wt = jax.nn.softplus(jnp.concatenate(heads, axis=0))
    s = jnp.dot(phiq_ref[...], phi_wt.astype(jnp.bfloat16),
                preferred_element_type=jnp.float32)

    base = pid * _TILE
    gcols = base + jax.lax.broadcasted_iota(jnp.int32, s.shape, 1)
    s = jnp.where(gcols < num_labels, s, _NEG)

    tv, ti = _top5(s, base)

    mv = mv_ref[...]
    mi = mi_ref[...]
    lanes = jax.lax.broadcasted_iota(jnp.int32, mv.shape, 1)
    for r in range(_TOPK):
        mv = jnp.where(lanes == _TOPK + r, tv[r], mv)
        mi = jnp.where(lanes == _TOPK + r, ti[r], mi)

    rvals, ridxs = [], []
    for _ in range(_TOPK):
        m = jnp.max(mv, axis=1, keepdims=True)
        am = jnp.argmax(mv, axis=1).astype(jnp.int32).reshape(-1, 1)
        rvals.append(m)
        ridxs.append(jnp.sum(jnp.where(lanes == am, mi, 0), axis=1,
                             keepdims=True))
        mv = jnp.where(lanes == am, _NEG, mv)

    new_mv = jnp.full(mv.shape, _NEG, jnp.float32)
    new_mi = jnp.zeros(mi.shape, jnp.int32)
    for r in range(_TOPK):
        new_mv = jnp.where(lanes == r, rvals[r], new_mv)
        new_mi = jnp.where(lanes == r, ridxs[r], new_mi)
    mv_ref[...] = new_mv
    mi_ref[...] = new_mi

    @pl.when(pid == num_tiles - 1)
    def _emit():
        vals_ref[...] = jnp.concatenate(rvals, axis=1)
        idxs_ref[...] = jnp.concatenate(ridxs, axis=1)


def kernel(indices, mask, embed_table, kernel, proj, k):
    del k
    embeds = jnp.take(embed_table, indices, axis=0)
    sum_embeds = jnp.sum(embeds * mask[:, :, None], axis=1)

    num_labels = kernel.shape[1]
    num_tiles = pl.cdiv(num_labels, _TILE)
    projt = jnp.transpose(proj, (0, 2, 1))

    fused = functools.partial(_fused_kernel, num_tiles=num_tiles,
                              num_labels=num_labels)
    vals, idxs = pl.pallas_call(
        fused,
        grid=(num_tiles,),
        in_specs=[
            pl.BlockSpec((_B, _D), lambda i: (0, 0)),
            pl.BlockSpec((_B, _L), lambda i: (0, 0)),
            pl.BlockSpec((_D, _TILE), lambda i: (0, i)),
            pl.BlockSpec((_H, _HD, _M), lambda i: (0, 0, 0)),
            pl.BlockSpec((_H, _M, _HD), lambda i: (0, 0, 0)),
        ],
        out_specs=[
            pl.BlockSpec((_B, _TOPK), lambda i: (0, 0)),
            pl.BlockSpec((_B, _TOPK), lambda i: (0, 0)),
        ],
        out_shape=[
            jax.ShapeDtypeStruct((_B, _TOPK), jnp.float32),
            jax.ShapeDtypeStruct((_B, _TOPK), jnp.int32),
        ],
        scratch_shapes=[
            pltpu.VMEM((_B, _F), jnp.bfloat16),
            pltpu.VMEM((_B, 128), jnp.float32),
            pltpu.VMEM((_B, 128), jnp.int32),
        ],
    )(sum_embeds, mask, kernel, proj, projt)
    return vals, idxs


# speedup vs baseline: 1.1789x; 1.1789x over previous
import functools

import jax
import jax.numpy as jnp
from jax.experimental import pallas as pl
from jax.experimental.pallas import tpu as pltpu

_B = 1024
_D = 64
_L = 50
_H = 4
_HD = 16
_M = 64
_F = _H * _M
_TOPK = 5
_TILE = 2048
_NEG = -1e30


def _normalize_rows(x):
    sq = x * x
    v = sq
    w = _D
    while w > 1:
        w //= 2
        v = v[:, :w] + v[:, w : 2 * w]
    n = jnp.sqrt(v)
    return x / jnp.maximum(n, 1e-4)


def _normalize_cols(x):
    sq = x * x
    v = sq
    w = _D
    while w > 1:
        w //= 2
        v = v[:w, :] + v[w : 2 * w, :]
    n = jnp.sqrt(v)
    return x / jnp.maximum(n, 1e-4)


def _performer(x, proj):
    outs = []
    for h in range(_H):
        xh = x[:, h * _HD : (h + 1) * _HD].astype(jnp.bfloat16)
        ph = proj[h].astype(jnp.bfloat16)
        outs.append(jnp.dot(xh, ph, preferred_element_type=jnp.float32))
    return jax.nn.softplus(jnp.concatenate(outs, axis=1))


def _top5(s, idx_off):
    colsf = jax.lax.broadcasted_iota(
        jnp.int32, s.shape, 1).astype(jnp.float32)
    vals, idxs = [], []
    for _ in range(_TOPK):
        m = jnp.max(s, axis=1, keepdims=True)
        candf = jnp.where(s == m, colsf, jnp.float32(3e9))
        amf = jnp.min(candf, axis=1, keepdims=True)
        vals.append(m)
        idxs.append(amf.astype(jnp.int32) + idx_off)
        s = jnp.where(colsf == amf, _NEG, s)
    return vals, idxs


def _fused_kernel(sums_ref, mask_ref, k_ref, proj_ref, projt_ref,
                  vals_ref, idxs_ref,
                  phiq_ref, mv_ref, mi_ref, *, num_tiles, num_labels):
    pid = pl.program_id(0)

    @pl.when(pid == 0)
    def _init():
        denom = jnp.maximum(
            jnp.sum(mask_ref[...], axis=1, keepdims=True), 1.0)
        q = _normalize_rows(sums_ref[...] / denom)
        phiq_ref[...] = _performer(q, proj_ref[...]).astype(jnp.bfloat16)
        mv_ref[...] = jnp.full(mv_ref.shape, _NEG, jnp.float32)
        mi_ref[...] = jnp.zeros(mi_ref.shape, jnp.int32)

    wn = _normalize_cols(k_ref[...])
    heads = []
    for h in range(_H):
        wh = wn[h * _HD : (h + 1) * _HD, :].astype(jnp.bfloat16)
        pt = projt_ref[h].astype(jnp.bfloat16)
        heads.append(jnp.dot(pt, wh, preferred_element_type=jnp.float32))
    phi_wt = jax.nn.softplus(jnp.concatenate(heads, axis=0))
    s = jnp.dot(phiq_ref[...], phi_wt.astype(jnp.bfloat16),
                preferred_element_type=jnp.float32)

    base = pid * _TILE
    gcols = base + jax.lax.broadcasted_iota(jnp.int32, s.shape, 1)
    s = jnp.where(gcols < num_labels, s, _NEG)

    tv, ti = _top5(s, base)

    mv = mv_ref[...]
    mi = mi_ref[...]
    lanes = jax.lax.broadcasted_iota(jnp.int32, mv.shape, 1)
    lanesf = lanes.astype(jnp.float32)
    for r in range(_TOPK):
        mv = jnp.where(lanes == _TOPK + r, tv[r], mv)
        mi = jnp.where(lanes == _TOPK + r, ti[r], mi)

    rvals, ridxs = [], []
    for _ in range(_TOPK):
        m = jnp.max(mv, axis=1, keepdims=True)
        candf = jnp.where(mv == m, lanesf, jnp.float32(3e9))
        amf = jnp.min(candf, axis=1, keepdims=True)
        am = amf.astype(jnp.int32)
        rvals.append(m)
        ridxs.append(jnp.sum(jnp.where(lanes == am, mi, 0), axis=1,
                             keepdims=True))
        mv = jnp.where(lanesf == amf, _NEG, mv)

    new_mv = jnp.full(mv.shape, _NEG, jnp.float32)
    new_mi = jnp.zeros(mi.shape, jnp.int32)
    for r in range(_TOPK):
        new_mv = jnp.where(lanes == r, rvals[r], new_mv)
        new_mi = jnp.where(lanes == r, ridxs[r], new_mi)
    mv_ref[...] = new_mv
    mi_ref[...] = new_mi

    @pl.when(pid == num_tiles - 1)
    def _emit():
        vals_ref[...] = jnp.concatenate(rvals, axis=1)
        idxs_ref[...] = jnp.concatenate(ridxs, axis=1)


def kernel(indices, mask, embed_table, kernel, proj, k):
    del k
    embeds = jnp.take(embed_table, indices, axis=0)
    sum_embeds = jnp.sum(embeds * mask[:, :, None], axis=1)

    num_labels = kernel.shape[1]
    num_tiles = pl.cdiv(num_labels, _TILE)
    projt = jnp.transpose(proj, (0, 2, 1))

    fused = functools.partial(_fused_kernel, num_tiles=num_tiles,
                              num_labels=num_labels)
    vals, idxs = pl.pallas_call(
        fused,
        grid=(num_tiles,),
        in_specs=[
            pl.BlockSpec((_B, _D), lambda i: (0, 0)),
            pl.BlockSpec((_B, _L), lambda i: (0, 0)),
            pl.BlockSpec((_D, _TILE), lambda i: (0, i)),
            pl.BlockSpec((_H, _HD, _M), lambda i: (0, 0, 0)),
            pl.BlockSpec((_H, _M, _HD), lambda i: (0, 0, 0)),
        ],
        out_specs=[
            pl.BlockSpec((_B, _TOPK), lambda i: (0, 0)),
            pl.BlockSpec((_B, _TOPK), lambda i: (0, 0)),
        ],
        out_shape=[
            jax.ShapeDtypeStruct((_B, _TOPK), jnp.float32),
            jax.ShapeDtypeStruct((_B, _TOPK), jnp.int32),
        ],
        scratch_shapes=[
            pltpu.VMEM((_B, _F), jnp.bfloat16),
            pltpu.VMEM((_B, 128), jnp.float32),
            pltpu.VMEM((_B, 128), jnp.int32),
        ],
    )(sum_embeds, mask, kernel, proj, projt)
    return vals, idxs
